# Initial kernel scaffold; baseline (speedup 1.0000x reference)
import jax
import jax.numpy as jnp
from jax.experimental import pallas as pl


def kernel(topk_values, topk_indices):
    raise NotImplementedError("write your pallas kernel here")


# speedup vs baseline: 8.6506x; 8.6506x over previous
import functools

import jax
import jax.numpy as jnp
from jax import lax
from jax.experimental import pallas as pl
from jax.experimental.pallas import tpu as pltpu
from jax.experimental.pallas import tpu_sc as plsc

B, S, TOPK, VOCAB = 16, 16, 4096, 50257
R = B * S
NC, NS, L = 2, 16, 16
NW = NC * NS
ROWS_PER_W = R // NW
ROWPAD = ((VOCAB + L - 1) // L) * L
NFILL = ROWPAD // L
NSCAT = TOPK // L


def _prep_body(v_ref, logv_ref, floor_ref):
    v = v_ref[...]
    logv_ref[...] = jnp.log(v + 1e-40)
    pmass = jnp.sum(v, axis=1)
    rem = jnp.clip(1.0 - pmass, 1e-40, 1.0)
    fl = jnp.log(rem / (VOCAB - TOPK))
    floor_ref[...] = jnp.broadcast_to(fl[:, None], (R, L))


_prep = pl.pallas_call(
    _prep_body,
    out_shape=[
        jax.ShapeDtypeStruct((R, TOPK), jnp.float32),
        jax.ShapeDtypeStruct((R, L), jnp.float32),
    ],
)


@functools.partial(
    pl.kernel,
    out_type=jax.ShapeDtypeStruct((R, VOCAB), jnp.float32),
    mesh=plsc.VectorSubcoreMesh(core_axis_name="c", subcore_axis_name="s"),
    compiler_params=pltpu.CompilerParams(
        needs_layout_passes=False, use_tc_tiling_on_sc=False),
    scratch_types=[
        pltpu.VMEM((ROWPAD,), jnp.float32),
        pltpu.VMEM((TOPK,), jnp.int32),
        pltpu.VMEM((TOPK,), jnp.float32),
        pltpu.VMEM((L,), jnp.float32),
    ],
)
def _sc_scatter(logv_hbm, idx_hbm, floor_hbm, out_hbm,
                rowbuf, idxbuf, valbuf, floorv):
    wid = lax.axis_index("s") * NC + lax.axis_index("c")
    r0 = wid * ROWS_PER_W
    for r in range(ROWS_PER_W):
        row = r0 + r
        pltpu.sync_copy(idx_hbm.at[row], idxbuf)
        pltpu.sync_copy(logv_hbm.at[row], valbuf)
        pltpu.sync_copy(floor_hbm.at[row], floorv)
        splat = floorv[...]

        def fill(i, carry):
            rowbuf[pl.ds(i * L, L)] = splat
            return carry

        lax.fori_loop(0, NFILL, fill, None)

        def scat(j, carry):
            iv = idxbuf[pl.ds(j * L, L)]
            vv = valbuf[pl.ds(j * L, L)]
            plsc.store_scatter(rowbuf, [iv], vv)
            return carry

        lax.fori_loop(0, NSCAT, scat, None)
        pltpu.sync_copy(rowbuf.at[pl.ds(0, VOCAB)], out_hbm.at[row])


def kernel(topk_values, topk_indices):
    v = topk_values.reshape(R, TOPK)
    idx = topk_indices.reshape(R, TOPK)
    logv, floor = _prep(v)
    out = _sc_scatter(logv, idx, floor)
    return out.reshape(B, S, VOCAB)


# speedup vs baseline: 43.8068x; 5.0640x over previous
import functools

import jax
import jax.numpy as jnp
from jax import lax
from jax.experimental import pallas as pl
from jax.experimental.pallas import tpu as pltpu
from jax.experimental.pallas import tpu_sc as plsc

B, S, TOPK, VOCAB = 16, 16, 4096, 50257
R = B * S
NC, NS, L = 2, 16, 16
NW = NC * NS
ROWS_PER_W = R // NW
ROWPAD = 50304
NFILL = ROWPAD // L
NSCAT = TOPK // L


def _prep_body(v_ref, logv_ref, floor_ref):
    v = v_ref[...]
    logv_ref[...] = jnp.log(v + 1e-40)
    pmass = jnp.sum(v, axis=-1)
    rem = jnp.clip(1.0 - pmass, 1e-40, 1.0)
    fl = jnp.log(rem / (VOCAB - TOPK))
    floor_ref[...] = jnp.broadcast_to(fl[:, :, None], (B, S, L))


_prep = pl.pallas_call(
    _prep_body,
    out_shape=[
        jax.ShapeDtypeStruct((B, S, TOPK), jnp.float32),
        jax.ShapeDtypeStruct((B, S, L), jnp.float32),
    ],
)


@functools.partial(
    pl.kernel,
    out_type=jax.ShapeDtypeStruct((R, ROWPAD), jnp.float32),
    mesh=plsc.VectorSubcoreMesh(core_axis_name="c", subcore_axis_name="s"),
    compiler_params=pltpu.CompilerParams(
        needs_layout_passes=False, use_tc_tiling_on_sc=False),
    scratch_types=[
        pltpu.VMEM((ROWPAD,), jnp.float32),
        pltpu.VMEM((ROWPAD,), jnp.float32),
        pltpu.VMEM((4, TOPK), jnp.int32),
        pltpu.VMEM((2, TOPK), jnp.float32),
        pltpu.VMEM((ROWS_PER_W, L), jnp.float32),
        pltpu.SemaphoreType.DMA,
        pltpu.SemaphoreType.DMA,
        pltpu.SemaphoreType.DMA((4,)),
        pltpu.SemaphoreType.DMA((2,)),
    ],
)
def _sc_scatter(logv_hbm, idx_hbm, floor_hbm, out_hbm,
                rb0, rb1, idxbuf, valbuf, floorbuf,
                sout0, sout1, sidx, sval):
    wid = lax.axis_index("s") * NC + lax.axis_index("c")
    b = wid >> 1
    s0 = (wid & 1) * ROWS_PER_W
    rbs = (rb0, rb1)
    souts = (sout0, sout1)

    pltpu.sync_copy(floor_hbm.at[b, pl.ds(s0, ROWS_PER_W)], floorbuf)

    def full_fill(rb, splat):
        def fill(i, carry):
            rb[pl.ds(i * L, L)] = splat
            return carry
        lax.fori_loop(0, NFILL, fill, None, unroll=8)

    def scat_vals(rb, q, p):
        def scat(j, carry):
            iv = idxbuf[q, pl.ds(j * L, L)]
            vv = valbuf[p, pl.ds(j * L, L)]
            plsc.store_scatter(rb, [iv], vv)
            return carry
        lax.fori_loop(0, NSCAT, scat, None, unroll=8)

    def scat_reset(rb, q, splat):
        def scat(j, carry):
            iv = idxbuf[q, pl.ds(j * L, L)]
            plsc.store_scatter(rb, [iv], splat)
            return carry
        lax.fori_loop(0, NSCAT, scat, None, unroll=8)

    cp_idx = [None] * 4
    cp_val = [None] * 2
    cp_out = [None] * 2
    for r in range(2):
        cp_idx[r] = pltpu.async_copy(
            idx_hbm.at[b, s0 + r], idxbuf.at[r], sidx.at[r])
        cp_val[r] = pltpu.async_copy(
            logv_hbm.at[b, s0 + r], valbuf.at[r], sval.at[r])

    for r in range(ROWS_PER_W):
        p = r % 2
        q = r % 4
        rb = rbs[p]
        splat = floorbuf[r]
        if r >= 2:
            cp_out[p].wait()
            prev = floorbuf[r - 2]
            same = jnp.max(jnp.abs(splat - prev)) == 0.0
            lax.cond(same,
                     lambda: scat_reset(rb, (r - 2) % 4, splat),
                     lambda: full_fill(rb, splat))
        else:
            full_fill(rb, splat)
        if r + 1 < ROWS_PER_W:
            nq = (r + 1) % 4
            cp_idx[nq] = pltpu.async_copy(
                idx_hbm.at[b, s0 + r + 1], idxbuf.at[nq], sidx.at[nq])
            cp_val[1 - p] = pltpu.async_copy(
                logv_hbm.at[b, s0 + r + 1], valbuf.at[1 - p], sval.at[1 - p])
        cp_idx[q].wait()
        cp_val[p].wait()
        scat_vals(rb, q, p)
        cp_out[p] = pltpu.async_copy(rb, out_hbm.at[wid * ROWS_PER_W + r],
                                     souts[p])
    cp_out[0].wait()
    cp_out[1].wait()


def kernel(topk_values, topk_indices):
    logv, floor = _prep(topk_values)
    out = _sc_scatter(logv, topk_indices, floor)
    return out.reshape(B, S, ROWPAD)[:, :, :VOCAB]
